# baseline (speedup 1.0000x reference)
import functools

import jax
import jax.numpy as jnp
from jax.experimental import pallas as pl
from jax.experimental.pallas import tpu as pltpu

_N = 20000
_NPAD = 20480
_ROWS = _NPAD // 128
_G = 64
_POS = 32
_NEG = 96
_K = _POS + _NEG

import numpy as np

_NEG_INF = np.float32(-np.inf)
_I32_MIN = np.int32(-(2 ** 31))
_I32_MAX = np.int32(2 ** 31 - 1)


def _orderkey(x):
    b = jax.lax.bitcast_convert_type(x, jnp.int32)
    return jnp.where(b < 0, b ^ jnp.int32(0x7FFFFFFF), b)


def _iou_body(boxes_tr_ref, gt_ref, miou_ref, ga_ref):
    b0 = boxes_tr_ref[0]
    b1 = boxes_tr_ref[1]
    b2 = boxes_tr_ref[2]
    b3 = boxes_tr_ref[3]
    area = (b2 - b0) * (b3 - b1)

    def body(g, carry):
        mi, ga = carry
        g0 = gt_ref[0, g]
        g1 = gt_ref[1, g]
        g2 = gt_ref[2, g]
        g3 = gt_ref[3, g]
        ty = jnp.maximum(b0, g0)
        tx = jnp.maximum(b1, g1)
        by = jnp.minimum(b2, g2)
        bx = jnp.minimum(b3, g3)
        inter = ((by - ty) * (bx - tx)) * jnp.where(
            (ty < by) & (tx < bx), jnp.float32(1.0), jnp.float32(0.0)
        )
        garea = (g2 - g0) * (g3 - g1)
        iou = inter / (area + garea - inter)
        better = iou > mi
        mi = jnp.where(better, iou, mi)
        ga = jnp.where(better, g, ga)
        return mi, ga

    mi0 = jnp.full((_ROWS, 128), _NEG_INF, jnp.float32)
    ga0 = jnp.zeros((_ROWS, 128), jnp.int32)
    mi, ga = jax.lax.fori_loop(0, _G, body, (mi0, ga0))
    miou_ref[...] = mi
    ga_ref[...] = ga


def _select_body(miou2d_ref, miou_col_ref, ga_col_ref, boxes_ref, gt_ref,
                 roi_ref, gtn_ref, label_ref, loc_ref):
    lin = (jax.lax.broadcasted_iota(jnp.int32, (_ROWS, 128), 0) * 128
           + jax.lax.broadcasted_iota(jnp.int32, (_ROWS, 128), 1))
    mi = jnp.where(lin < _N, miou2d_ref[...], _NEG_INF)
    kp = _orderkey(jnp.where(mi >= 0.5, mi, _NEG_INF))
    kn = _orderkey(jnp.where(mi < 0.5, mi, _NEG_INF))

    def make_step(k_off):
        def step(k, keys):
            m = jnp.max(keys)
            idx = jnp.min(jnp.where(keys == m, lin, _I32_MAX))
            keys = jnp.where(lin == idx, _I32_MIN, keys)
            mi_v = miou_col_ref[pl.ds(idx, 1), :]
            ga_s = jnp.max(ga_col_ref[pl.ds(idx, 1), :])
            o = k_off + k
            roi_ref[pl.ds(o, 1), :] = boxes_ref[pl.ds(idx, 1), :]
            gtn_ref[pl.ds(o, 1), :] = gt_ref[pl.ds(ga_s, 1), :]
            label_ref[pl.ds(o, 1), :] = (mi_v >= 0.5).astype(jnp.int32)
            return keys

        return step

    jax.lax.fori_loop(0, _POS, make_step(0), kp)
    jax.lax.fori_loop(0, _NEG, make_step(_POS), kn)

    r = roi_ref[...]
    g = gtn_ref[...]
    h = r[:, 2:3] - r[:, 0:1]
    w = r[:, 3:4] - r[:, 1:2]
    dy = (g[:, 2:3] + g[:, 0:1] - r[:, 2:3] - r[:, 0:1]) / 2.0 / h
    dx = (g[:, 3:4] + g[:, 2:3] - r[:, 3:4] - r[:, 2:3]) / 2.0 / w
    dh = jnp.log(jnp.maximum(h - g[:, 2:3] + g[:, 0:1], jnp.float32(1e-6)))
    dw = jnp.log(jnp.maximum(w - g[:, 3:4] + g[:, 1:2], jnp.float32(1e-6)))
    loc_ref[...] = jnp.concatenate([dy, dx, dh, dw], axis=1)


@jax.jit
def kernel(boxes, gt_bboxes):
    boxes_p = jnp.pad(boxes, ((0, _NPAD - _N), (0, 0)))
    boxes_tr = boxes_p.T.reshape(4, _ROWS, 128)
    gt_t = gt_bboxes.T

    miou2d, ga2d = pl.pallas_call(
        _iou_body,
        out_shape=[
            jax.ShapeDtypeStruct((_ROWS, 128), jnp.float32),
            jax.ShapeDtypeStruct((_ROWS, 128), jnp.int32),
        ],
        in_specs=[
            pl.BlockSpec(memory_space=pltpu.VMEM),
            pl.BlockSpec(memory_space=pltpu.SMEM),
        ],
    )(boxes_tr, gt_t)

    miou_col = miou2d.reshape(_NPAD, 1)
    ga_col = ga2d.reshape(_NPAD, 1)

    roi, gtn, label, loc = pl.pallas_call(
        _select_body,
        out_shape=[
            jax.ShapeDtypeStruct((_K, 4), jnp.float32),
            jax.ShapeDtypeStruct((_K, 4), jnp.float32),
            jax.ShapeDtypeStruct((_K, 1), jnp.int32),
            jax.ShapeDtypeStruct((_K, 4), jnp.float32),
        ],
        in_specs=[pl.BlockSpec(memory_space=pltpu.VMEM)] * 5,
    )(miou2d, miou_col, ga_col, boxes_p, gt_bboxes)

    return roi, gtn, label.reshape(_K), loc
